# Initial kernel scaffold; baseline (speedup 1.0000x reference)
import jax
import jax.numpy as jnp
from jax.experimental import pallas as pl


def kernel(dwi_x, t2_x, dwi_w0, dwi_b0, dwi_w1, dwi_b1, dwi_w2, dwi_b2, dwi_w3, dwi_b3, dwi_w4, dwi_b4, dwi_w5, dwi_b5, dwi_w6, dwi_b6, dwi_w7, dwi_b7, t2_w0, t2_b0, t2_w1, t2_b1, t2_w2, t2_b2, t2_w3, t2_b3, t2_w4, t2_b4, t2_w5, t2_b5, t2_w6, t2_b6, t2_w7, t2_b7, lin_w, lin_b):
    raise NotImplementedError("write your pallas kernel here")


# speedup vs baseline: 2.0894x; 2.0894x over previous
import functools

import jax
import jax.numpy as jnp
from jax.experimental import pallas as pl
from jax.experimental.pallas import tpu as pltpu


def _conv_body(x_ref, w_ref, b_ref, o_ref, *, cin, cout, d_out, hw, wf,
               out_off, zero_ends, relu):
    m = hw - 2 * wf - 2
    for d in range(d_out):
        acc = b_ref[...] + jnp.zeros((cout, m), jnp.float32)
        for t in range(9):
            kh, kw = divmod(t, 3)
            xs = x_ref[pl.ds(d * cin, 3 * cin), pl.ds(kh * wf + kw, m)]
            acc = acc + jnp.dot(w_ref[t], xs,
                                preferred_element_type=jnp.float32)
        if relu:
            acc = jnp.maximum(acc, 0.0)
        o_ref[pl.ds((d + out_off) * cout, cout), pl.ds(0, m)] = acc
    if zero_ends:
        z = jnp.zeros((cout, hw), jnp.float32)
        o_ref[pl.ds(0, cout), :] = z
        o_ref[pl.ds((d_out + 1) * cout, cout), :] = z


def _conv(x, w, b, *, d_in, cin, cout, hw, wf, pad_out, relu):
    n = x.shape[0]
    d_out = d_in - 2
    out_off = 1 if pad_out else 0
    d_frame = d_out + 2 * out_off
    w9 = jnp.transpose(w, (3, 4, 0, 2, 1)).reshape(9, cout, 3 * cin)
    b2 = b.reshape(cout, 1)
    body = functools.partial(
        _conv_body, cin=cin, cout=cout, d_out=d_out, hw=hw, wf=wf,
        out_off=out_off, zero_ends=pad_out, relu=relu)
    return pl.pallas_call(
        body,
        out_shape=jax.ShapeDtypeStruct((n, d_frame * cout, hw), jnp.float32),
        grid=(n,),
        in_specs=[
            pl.BlockSpec((None, d_in * cin, hw), lambda i: (i, 0, 0)),
            pl.BlockSpec((9, cout, 3 * cin), lambda i: (0, 0, 0)),
            pl.BlockSpec((cout, 1), lambda i: (0, 0)),
        ],
        out_specs=pl.BlockSpec((None, d_frame * cout, hw), lambda i: (i, 0, 0)),
        compiler_params=pltpu.CompilerParams(
            dimension_semantics=("parallel",),
            vmem_limit_bytes=60 * 1024 * 1024),
    )(x, w9, b2)


def _pool(y, *, n, d, c, h, w, vh, vw, pool_d):
    y5 = y.reshape(n, d, c, h, w)
    ph, pw = (vh // 2) * 2, (vw // 2) * 2
    if pool_d == 2:
        pd = (d // 2) * 2
        y5 = y5[:, :pd, :, :ph, :pw]
        y5 = y5.reshape(n, pd // 2, 2, c, ph // 2, 2, pw // 2, 2)
        y5 = y5.max(axis=(2, 5, 7))
        do = pd // 2
    else:
        y5 = y5[:, :, :, :ph, :pw]
        y5 = y5.reshape(n, d, c, ph // 2, 2, pw // 2, 2)
        y5 = y5.max(axis=(4, 6))
        do = d
    y5 = jnp.pad(y5, ((0, 0), (1, 1), (0, 0), (0, 0), (0, 0)))
    return (y5.reshape(n, (do + 2) * c, (ph // 2) * (pw // 2)),
            do, ph // 2, pw // 2)


def _tower(x5, params):
    n, c, d, h, w = x5.shape
    xt = jnp.transpose(x5, (0, 2, 1, 3, 4))
    xt = jnp.pad(xt, ((0, 0), (0, 0), (0, 8 - c), (0, 0), (0, 0)))
    x = xt.reshape(n, d * 8, h * w)

    (w0, b0), (w1, b1), (w2, b2), (w3, b3), (w4, b4), (w5, b5), (w6, b6), \
        (w7, b7) = params
    w0p = jnp.pad(w0, ((0, 0), (0, 8 - c), (0, 0), (0, 0), (0, 0)))

    hw, wf = h * w, w
    x = _conv(x, w0p, b0, d_in=d, cin=8, cout=8, hw=hw, wf=wf,
              pad_out=False, relu=True)
    d0 = d - 2
    x = _conv(x, w1, b1, d_in=d0, cin=8, cout=16, hw=hw, wf=wf,
              pad_out=False, relu=True)
    d1 = d0 - 2
    x, d2, h2, w2_ = _pool(x, n=n, d=d1, c=16, h=h, w=w, vh=h - 4, vw=w - 4,
                           pool_d=2)

    hw2, wf2 = h2 * w2_, w2_
    x = _conv(x, w2, b2, d_in=d2 + 2, cin=16, cout=16, hw=hw2, wf=wf2,
              pad_out=True, relu=True)
    x = _conv(x, w3, b3, d_in=d2 + 2, cin=16, cout=32, hw=hw2, wf=wf2,
              pad_out=False, relu=True)
    x, d3, h3, w3_ = _pool(x, n=n, d=d2, c=32, h=h2, w=w2_, vh=h2 - 4,
                           vw=w2_ - 4, pool_d=1)

    hw3, wf3 = h3 * w3_, w3_
    x = _conv(x, w4, b4, d_in=d3 + 2, cin=32, cout=64, hw=hw3, wf=wf3,
              pad_out=True, relu=True)
    x = _conv(x, w5, b5, d_in=d3 + 2, cin=64, cout=32, hw=hw3, wf=wf3,
              pad_out=False, relu=True)
    x, d4, h4, w4_ = _pool(x, n=n, d=d3, c=32, h=h3, w=w3_, vh=h3 - 4,
                           vw=w3_ - 4, pool_d=1)

    hw4, wf4 = h4 * w4_, w4_
    x = _conv(x, w6, b6, d_in=d4 + 2, cin=32, cout=64, hw=hw4, wf=wf4,
              pad_out=True, relu=True)
    x = _conv(x, w7, b7, d_in=d4 + 2, cin=64, cout=32, hw=hw4, wf=wf4,
              pad_out=False, relu=False)

    vh, vw = h4 - 4, w4_ - 4
    y5 = x.reshape(n, d4, 32, h4, w4_)[:, :, :, :vh, :vw]
    y5 = jnp.transpose(y5, (0, 1, 3, 4, 2))
    return y5.reshape(n, d4 * vh * vw, 32)


def _head_body(a_ref, t_ref, wa_ref, wt_ref, b_ref, soft_ref, out_ref):
    sa = jnp.mean(a_ref[...], axis=1)
    st = jnp.mean(t_ref[...], axis=1)
    logits = (jnp.sum(sa * wa_ref[...], axis=1, keepdims=True)
              + jnp.sum(st * wt_ref[...], axis=1, keepdims=True)
              + b_ref[...])
    out_ref[...] = logits
    soft_ref[...] = jnp.ones_like(logits)


def kernel(dwi_x, t2_x,
           dwi_w0, dwi_b0, dwi_w1, dwi_b1, dwi_w2, dwi_b2, dwi_w3, dwi_b3,
           dwi_w4, dwi_b4, dwi_w5, dwi_b5, dwi_w6, dwi_b6, dwi_w7, dwi_b7,
           t2_w0, t2_b0, t2_w1, t2_b1, t2_w2, t2_b2, t2_w3, t2_b3,
           t2_w4, t2_b4, t2_w5, t2_b5, t2_w6, t2_b6, t2_w7, t2_b7,
           lin_w, lin_b):
    p_dwi = [(dwi_w0, dwi_b0), (dwi_w1, dwi_b1), (dwi_w2, dwi_b2),
             (dwi_w3, dwi_b3), (dwi_w4, dwi_b4), (dwi_w5, dwi_b5),
             (dwi_w6, dwi_b6), (dwi_w7, dwi_b7)]
    p_t2 = [(t2_w0, t2_b0), (t2_w1, t2_b1), (t2_w2, t2_b2), (t2_w3, t2_b3),
            (t2_w4, t2_b4), (t2_w5, t2_b5), (t2_w6, t2_b6), (t2_w7, t2_b7)]
    f_dwi = _tower(dwi_x, p_dwi)
    f_t2 = _tower(t2_x, p_t2)
    n = f_dwi.shape[0]
    soft, out = pl.pallas_call(
        _head_body,
        out_shape=(jax.ShapeDtypeStruct((n, 1), jnp.float32),
                   jax.ShapeDtypeStruct((n, 1), jnp.float32)),
    )(f_dwi, f_t2, lin_w[:, :32], lin_w[:, 32:], lin_b.reshape(1, 1))
    return soft, out
